# baseline (speedup 1.0000x reference)
import functools

import jax
import jax.numpy as jnp
from jax.experimental import pallas as pl
from jax.experimental.pallas import tpu as pltpu

_VOCAB = 8192
_B, _T = 16, 512
_N = _B * _T
_R = 16
_GRID = _N // _R


def _copy_body(idx_ref, *refs):
    in_refs = refs[:_R]
    out_ref = refs[_R]
    for r in range(_R):
        out_ref[r, :] = in_refs[r][0, 0, :]


def _row_spec(r):
    return pl.BlockSpec((1, 1, _VOCAB), lambda i, idx, _r=r: (idx[i * _R + _r], 0, 0))


_grid_spec = pltpu.PrefetchScalarGridSpec(
    num_scalar_prefetch=1,
    grid=(_GRID,),
    in_specs=[_row_spec(r) for r in range(_R)],
    out_specs=pl.BlockSpec((_R, _VOCAB), lambda i, idx: (i, 0)),
)

_tc_gather = pl.pallas_call(
    _copy_body,
    grid_spec=_grid_spec,
    out_shape=jax.ShapeDtypeStruct((_N, _VOCAB), jnp.float32),
    compiler_params=pltpu.CompilerParams(
        dimension_semantics=("arbitrary",),
    ),
)


def kernel(X, table):
    idx = X.reshape(_N).astype(jnp.int32)
    t3 = table.reshape(_VOCAB, 1, _VOCAB)
    out = _tc_gather(idx, *([t3] * _R))
    return out.reshape(_B, _T, _VOCAB)
